# Initial kernel scaffold; baseline (speedup 1.0000x reference)
import jax
import jax.numpy as jnp
from jax.experimental import pallas as pl


def kernel(x, edge_index_rel0, edge_index_rel1, W0, b0, W1, b1, prelu_a):
    raise NotImplementedError("write your pallas kernel here")


# speedup vs baseline: 13.2512x; 13.2512x over previous
import functools

import jax
import jax.numpy as jnp
from jax import lax
from jax.experimental import pallas as pl
from jax.experimental.pallas import tpu as pltpu
from jax.experimental.pallas import tpu_sc as plsc

N = 10000
D = 128
E = 320000
NT = 10240
NC = 2
NS = 16
EPC = 128
EPT = E // NS
CH = 160
EPTP = CH * EPC
PADE = EPTP - EPT
G = 8
NB = CH // G

_mesh = plsc.VectorSubcoreMesh(core_axis_name="c", subcore_axis_name="s")


def _fill_f32(ref, rows, cols, value):
    v = jnp.full((16,), value, dtype=jnp.float32)
    nchunks = cols // 16

    def body(i, _):
        for k in range(nchunks):
            ref[i, pl.ds(16 * k, 16)] = v
        return 0

    lax.fori_loop(0, rows, body, 0)


@functools.partial(
    pl.kernel,
    out_type=(
        jax.ShapeDtypeStruct((NC, 2 * NT), jnp.float32),
        jax.ShapeDtypeStruct((NC, NT), jnp.float32),
    ),
    mesh=_mesh,
    scratch_types=[
        pltpu.VMEM((CH, EPC), jnp.int32),
        pltpu.VMEM((CH, EPC), jnp.int32),
        pltpu.VMEM((1, EPC), jnp.float32),
        pltpu.VMEM((1, 2 * NT // NS), jnp.float32),
        pltpu.VMEM_SHARED((2 * NT,), jnp.float32),
        pltpu.VMEM_SHARED((NT,), jnp.float32),
        pltpu.SemaphoreType.DMA,
        pltpu.SemaphoreType.DMA,
    ],
)
def _deg_kernel(src_hbm, dst_hbm, degsrc_hbm, degdst_hbm,
                src_v, dst_v, ones_v, zeros_v, shist, dhist, sem0, sem1):
    c = lax.axis_index("c")
    s = lax.axis_index("s")
    sh_slice = 2 * NT // NS
    dh_slice = NT // NS

    pltpu.sync_copy(src_hbm.at[c, s], src_v)
    pltpu.sync_copy(dst_hbm.at[c, s], dst_v)
    _fill_f32(ones_v, 1, EPC, 1.0)
    _fill_f32(zeros_v, 1, sh_slice, 0.0)

    pltpu.sync_copy(zeros_v.at[0], shist.at[pl.ds(s * sh_slice, sh_slice)])
    pltpu.sync_copy(zeros_v.at[0, pl.ds(0, dh_slice)],
                    dhist.at[pl.ds(s * dh_slice, dh_slice)])
    plsc.subcore_barrier()

    def body(j, _):
        a = pltpu.async_copy(ones_v.at[0], shist.at[src_v.at[j]], sem0,
                             add=True)
        b = pltpu.async_copy(ones_v.at[0], dhist.at[dst_v.at[j]], sem1,
                             add=True)
        a.wait()
        b.wait()
        return 0

    lax.fori_loop(0, CH, body, 0)
    plsc.subcore_barrier()

    pltpu.sync_copy(shist.at[pl.ds(s * sh_slice, sh_slice)],
                    degsrc_hbm.at[c, pl.ds(s * sh_slice, sh_slice)])
    pltpu.sync_copy(dhist.at[pl.ds(s * dh_slice, dh_slice)],
                    degdst_hbm.at[c, pl.ds(s * dh_slice, dh_slice)])


@functools.partial(
    pl.kernel,
    out_type=jax.ShapeDtypeStruct((2 * NT, D), jnp.float32),
    mesh=_mesh,
    scratch_types=[
        pltpu.VMEM((2, G, EPC), jnp.int32),
        pltpu.VMEM((2, G, EPC), jnp.int32),
        pltpu.VMEM((EPC, D), jnp.float32),
        pltpu.VMEM((EPC, D), jnp.float32),
        pltpu.VMEM_SHARED((NT, D), jnp.float32),
        pltpu.SemaphoreType.DMA,
        pltpu.SemaphoreType.DMA,
        pltpu.SemaphoreType.DMA,
        pltpu.SemaphoreType.DMA,
    ],
)
def _scatter_kernel(z_hbm, src_hbm, dst_hbm, agg_hbm,
                    srcblk, dstblk, buf0, buf1, acc,
                    sem0, sem1, sem_si, sem_di):
    c = lax.axis_index("c")
    s = lax.axis_index("s")
    rows_per_tile = NT // NS
    bufs = (buf0, buf1)
    sems = (sem0, sem1)

    def start_blk_load(q, p, sem_s, sem_d):
        pltpu.async_copy(src_hbm.at[c, s, pl.ds(G * q, G)], srcblk.at[p],
                         sem_s)
        pltpu.async_copy(dst_hbm.at[c, s, pl.ds(G * q, G)], dstblk.at[p],
                         sem_d)

    def wait_blk_load(q, p):
        pltpu.make_async_copy(src_hbm.at[c, s, pl.ds(G * q, G)],
                              srcblk.at[p], sem_si).wait()
        pltpu.make_async_copy(dst_hbm.at[c, s, pl.ds(G * q, G)],
                              dstblk.at[p], sem_di).wait()

    start_blk_load(0, 0, sem_si, sem_di)
    wait_blk_load(0, 0)
    start_blk_load(1, 1, sem_si, sem_di)

    _fill_f32(buf0, EPC, D, 0.0)
    for k in range(rows_per_tile // EPC):
        pltpu.sync_copy(
            buf0, acc.at[pl.ds(s * rows_per_tile + k * EPC, EPC)])
    plsc.subcore_barrier()

    pltpu.async_copy(z_hbm.at[srcblk.at[0, 0]], buf0, sem0)

    def gbody(u, _):
        for e in range(2):
            q = 2 * u + e
            for r in range(G):
                b = r % 2
                if r == G - 1:
                    @pl.when(q + 1 < NB)
                    def _():
                        wait_blk_load(q + 1, 1 - e)

                nxt_p = e if r < G - 1 else 1 - e
                nxt_r = (r + 1) % G

                @pl.when(G * q + r + 1 < CH)
                def _():
                    pltpu.async_copy(z_hbm.at[srcblk.at[nxt_p, nxt_r]],
                                     bufs[1 - b], sems[1 - b])

                pltpu.make_async_copy(z_hbm.at[srcblk.at[e, r]], bufs[b],
                                      sems[b]).wait()
                pltpu.sync_copy(bufs[b], acc.at[dstblk.at[e, r]], add=True)

            @pl.when(q + 2 < NB)
            def _():
                start_blk_load(q + 2, e, sem_si, sem_di)
        return 0

    lax.fori_loop(0, NB // 2, gbody, 0)
    plsc.subcore_barrier()

    pltpu.sync_copy(
        acc.at[pl.ds(s * rows_per_tile, rows_per_tile)],
        agg_hbm.at[pl.ds(c * NT + s * rows_per_tile, rows_per_tile)])


def _mask_rsqrt(d):
    return jnp.where(d > 0, lax.rsqrt(jnp.maximum(d, 1.0)), 0.0)


def _scale_body(x_ref, degsrc_ref, z_ref):
    x = x_ref[...]
    ns0 = _mask_rsqrt(degsrc_ref[pl.ds(0, N), :])
    ns1 = _mask_rsqrt(degsrc_ref[pl.ds(3 * NT, N), :])
    zeros = jnp.zeros((NT - N, D), jnp.float32)
    z_ref[pl.ds(0, N), :] = x * ns0
    z_ref[pl.ds(N, NT - N), :] = zeros
    z_ref[pl.ds(NT, N), :] = x * ns1
    z_ref[pl.ds(NT + N, NT - N), :] = zeros


def _out_body(agg_ref, degdst_ref, w0_ref, w1_ref, b0_ref, b1_ref, a_ref,
              h_ref):
    nd0 = _mask_rsqrt(degdst_ref[pl.ds(0, N), :])
    nd1 = _mask_rsqrt(degdst_ref[pl.ds(NT, N), :])
    m0 = jnp.dot(agg_ref[pl.ds(0, N), :], w0_ref[...],
                 preferred_element_type=jnp.float32)
    m1 = jnp.dot(agg_ref[pl.ds(NT, N), :], w1_ref[...],
                 preferred_element_type=jnp.float32)
    h = nd0 * m0 + nd1 * m1 + (b0_ref[...] + b1_ref[...])
    a = a_ref[0, 0]
    h_ref[...] = jnp.where(h > 0, h, a * h)


def _prep_idx(idx, bump):
    a = idx.astype(jnp.int32).reshape(NS, EPT) + jnp.int32(bump)
    t = jnp.arange(NS, dtype=jnp.int32)[:, None]
    p = jnp.arange(PADE, dtype=jnp.int32)[None, :]
    padvals = jnp.int32(N) + (t * 37 + p) % jnp.int32(NT - N) + jnp.int32(bump)
    return jnp.concatenate([a, padvals], axis=1).reshape(NS, CH, EPC)


def kernel(x, edge_index_rel0, edge_index_rel1, W0, b0, W1, b1, prelu_a):
    src_all = jnp.stack([_prep_idx(edge_index_rel0[0], 0),
                         _prep_idx(edge_index_rel1[0], NT)])
    dst_all = jnp.stack([_prep_idx(edge_index_rel0[1], 0),
                         _prep_idx(edge_index_rel1[1], 0)])

    degsrc, degdst = _deg_kernel(src_all, dst_all)

    z = pl.pallas_call(
        _scale_body,
        out_shape=jax.ShapeDtypeStruct((2 * NT, D), jnp.float32),
    )(x, degsrc.reshape(2 * 2 * NT, 1))

    agg = _scatter_kernel(z, src_all, dst_all)

    h = pl.pallas_call(
        _out_body,
        out_shape=jax.ShapeDtypeStruct((N, D), jnp.float32),
    )(agg, degdst.reshape(2 * NT, 1), W0, W1, b0.reshape(1, D),
      b1.reshape(1, D), prelu_a.reshape(1, 1))
    return h


# speedup vs baseline: 13.2531x; 1.0001x over previous
import functools

import jax
import jax.numpy as jnp
from jax import lax
from jax.experimental import pallas as pl
from jax.experimental.pallas import tpu as pltpu
from jax.experimental.pallas import tpu_sc as plsc

N = 10000
D = 128
E = 320000
NT = 10240
NC = 2
NS = 16
EPC = 128
EPT = E // NS
CH = 160
EPTP = CH * EPC
PADE = EPTP - EPT
G = 8
NB = CH // G

_mesh = plsc.VectorSubcoreMesh(core_axis_name="c", subcore_axis_name="s")


def _fill_f32(ref, rows, cols, value):
    v = jnp.full((16,), value, dtype=jnp.float32)
    nchunks = cols // 16

    def body(i, _):
        for k in range(nchunks):
            ref[i, pl.ds(16 * k, 16)] = v
        return 0

    lax.fori_loop(0, rows, body, 0)


@functools.partial(
    pl.kernel,
    out_type=(
        jax.ShapeDtypeStruct((NC, 2 * NT), jnp.float32),
        jax.ShapeDtypeStruct((NC, NT), jnp.float32),
    ),
    mesh=_mesh,
    scratch_types=[
        pltpu.VMEM((CH, EPC), jnp.int32),
        pltpu.VMEM((CH, EPC), jnp.int32),
        pltpu.VMEM((1, EPC), jnp.float32),
        pltpu.VMEM((1, 2 * NT // NS), jnp.float32),
        pltpu.VMEM_SHARED((2 * NT,), jnp.float32),
        pltpu.VMEM_SHARED((NT,), jnp.float32),
        pltpu.SemaphoreType.DMA,
        pltpu.SemaphoreType.DMA,
    ],
)
def _deg_kernel(src_hbm, dst_hbm, degsrc_hbm, degdst_hbm,
                src_v, dst_v, ones_v, zeros_v, shist, dhist, sem0, sem1):
    c = lax.axis_index("c")
    s = lax.axis_index("s")
    sh_slice = 2 * NT // NS
    dh_slice = NT // NS

    pltpu.sync_copy(src_hbm.at[c, s], src_v)
    pltpu.sync_copy(dst_hbm.at[c, s], dst_v)
    _fill_f32(ones_v, 1, EPC, 1.0)
    _fill_f32(zeros_v, 1, sh_slice, 0.0)

    pltpu.sync_copy(zeros_v.at[0], shist.at[pl.ds(s * sh_slice, sh_slice)])
    pltpu.sync_copy(zeros_v.at[0, pl.ds(0, dh_slice)],
                    dhist.at[pl.ds(s * dh_slice, dh_slice)])
    plsc.subcore_barrier()

    def body(j, _):
        a = pltpu.async_copy(ones_v.at[0], shist.at[src_v.at[j]], sem0,
                             add=True)
        b = pltpu.async_copy(ones_v.at[0], dhist.at[dst_v.at[j]], sem1,
                             add=True)
        a.wait()
        b.wait()
        return 0

    lax.fori_loop(0, CH, body, 0)
    plsc.subcore_barrier()

    pltpu.sync_copy(shist.at[pl.ds(s * sh_slice, sh_slice)],
                    degsrc_hbm.at[c, pl.ds(s * sh_slice, sh_slice)])
    pltpu.sync_copy(dhist.at[pl.ds(s * dh_slice, dh_slice)],
                    degdst_hbm.at[c, pl.ds(s * dh_slice, dh_slice)])


@functools.partial(
    pl.kernel,
    out_type=jax.ShapeDtypeStruct((2 * NT, D), jnp.float32),
    mesh=_mesh,
    scratch_types=[
        pltpu.VMEM((2, G, EPC), jnp.int32),
        pltpu.VMEM((2, G, EPC), jnp.int32),
        pltpu.VMEM((EPC, D), jnp.float32),
        pltpu.VMEM((EPC, D), jnp.float32),
        pltpu.VMEM_SHARED((NT, D), jnp.float32),
        pltpu.SemaphoreType.DMA,
        pltpu.SemaphoreType.DMA,
        pltpu.SemaphoreType.DMA,
        pltpu.SemaphoreType.DMA,
        pltpu.SemaphoreType.DMA,
        pltpu.SemaphoreType.DMA,
    ],
)
def _scatter_kernel(z_hbm, src_hbm, dst_hbm, agg_hbm,
                    srcblk, dstblk, buf0, buf1, acc,
                    sem0, sem1, ssem0, ssem1, sem_si, sem_di):
    c = lax.axis_index("c")
    s = lax.axis_index("s")
    rows_per_tile = NT // NS
    bufs = (buf0, buf1)
    sems = (sem0, sem1)

    def start_blk_load(q, p, sem_s, sem_d):
        pltpu.async_copy(src_hbm.at[c, s, pl.ds(G * q, G)], srcblk.at[p],
                         sem_s)
        pltpu.async_copy(dst_hbm.at[c, s, pl.ds(G * q, G)], dstblk.at[p],
                         sem_d)

    def wait_blk_load(q, p):
        pltpu.make_async_copy(src_hbm.at[c, s, pl.ds(G * q, G)],
                              srcblk.at[p], sem_si).wait()
        pltpu.make_async_copy(dst_hbm.at[c, s, pl.ds(G * q, G)],
                              dstblk.at[p], sem_di).wait()

    start_blk_load(0, 0, sem_si, sem_di)
    wait_blk_load(0, 0)
    start_blk_load(1, 1, sem_si, sem_di)

    _fill_f32(buf0, EPC, D, 0.0)
    for k in range(rows_per_tile // EPC):
        pltpu.sync_copy(
            buf0, acc.at[pl.ds(s * rows_per_tile + k * EPC, EPC)])
    plsc.subcore_barrier()

    pltpu.async_copy(z_hbm.at[srcblk.at[0, 0]], buf0, sem0)

    ssems = (ssem0, ssem1)

    def gbody(u, _):
        for e in range(2):
            q = 2 * u + e
            for r in range(G):
                b = r % 2
                jj = G * q + r
                if r == G - 1:
                    @pl.when(q + 1 < NB)
                    def _():
                        wait_blk_load(q + 1, 1 - e)

                nxt_p = e if r < G - 1 else 1 - e
                nxt_r = (r + 1) % G

                @pl.when(jj >= 1)
                def _():
                    pltpu.make_async_copy(bufs[1 - b],
                                          acc.at[dstblk.at[e, r]],
                                          ssems[1 - b]).wait()

                @pl.when(jj + 1 < CH)
                def _():
                    pltpu.async_copy(z_hbm.at[srcblk.at[nxt_p, nxt_r]],
                                     bufs[1 - b], sems[1 - b])

                pltpu.make_async_copy(z_hbm.at[srcblk.at[e, r]], bufs[b],
                                      sems[b]).wait()
                pltpu.async_copy(bufs[b], acc.at[dstblk.at[e, r]], ssems[b],
                                 add=True)

            @pl.when(q + 2 < NB)
            def _():
                start_blk_load(q + 2, e, sem_si, sem_di)
        return 0

    lax.fori_loop(0, NB // 2, gbody, 0)
    pltpu.make_async_copy(bufs[(CH - 1) % 2], acc.at[dstblk.at[1, G - 1]],
                          ssems[(CH - 1) % 2]).wait()
    plsc.subcore_barrier()

    pltpu.sync_copy(
        acc.at[pl.ds(s * rows_per_tile, rows_per_tile)],
        agg_hbm.at[pl.ds(c * NT + s * rows_per_tile, rows_per_tile)])


def _mask_rsqrt(d):
    return jnp.where(d > 0, lax.rsqrt(jnp.maximum(d, 1.0)), 0.0)


def _scale_body(x_ref, degsrc_ref, z_ref):
    x = x_ref[...]
    ns0 = _mask_rsqrt(degsrc_ref[pl.ds(0, N), :])
    ns1 = _mask_rsqrt(degsrc_ref[pl.ds(3 * NT, N), :])
    zeros = jnp.zeros((NT - N, D), jnp.float32)
    z_ref[pl.ds(0, N), :] = x * ns0
    z_ref[pl.ds(N, NT - N), :] = zeros
    z_ref[pl.ds(NT, N), :] = x * ns1
    z_ref[pl.ds(NT + N, NT - N), :] = zeros


def _out_body(agg_ref, degdst_ref, w0_ref, w1_ref, b0_ref, b1_ref, a_ref,
              h_ref):
    nd0 = _mask_rsqrt(degdst_ref[pl.ds(0, N), :])
    nd1 = _mask_rsqrt(degdst_ref[pl.ds(NT, N), :])
    m0 = jnp.dot(agg_ref[pl.ds(0, N), :], w0_ref[...],
                 preferred_element_type=jnp.float32)
    m1 = jnp.dot(agg_ref[pl.ds(NT, N), :], w1_ref[...],
                 preferred_element_type=jnp.float32)
    h = nd0 * m0 + nd1 * m1 + (b0_ref[...] + b1_ref[...])
    a = a_ref[0, 0]
    h_ref[...] = jnp.where(h > 0, h, a * h)


def _prep_idx(idx, bump):
    a = idx.astype(jnp.int32).reshape(NS, EPT) + jnp.int32(bump)
    t = jnp.arange(NS, dtype=jnp.int32)[:, None]
    p = jnp.arange(PADE, dtype=jnp.int32)[None, :]
    padvals = jnp.int32(N) + (t * 37 + p) % jnp.int32(NT - N) + jnp.int32(bump)
    return jnp.concatenate([a, padvals], axis=1).reshape(NS, CH, EPC)


def kernel(x, edge_index_rel0, edge_index_rel1, W0, b0, W1, b1, prelu_a):
    src_all = jnp.stack([_prep_idx(edge_index_rel0[0], 0),
                         _prep_idx(edge_index_rel1[0], NT)])
    dst_all = jnp.stack([_prep_idx(edge_index_rel0[1], 0),
                         _prep_idx(edge_index_rel1[1], 0)])

    degsrc, degdst = _deg_kernel(src_all, dst_all)

    z = pl.pallas_call(
        _scale_body,
        out_shape=jax.ShapeDtypeStruct((2 * NT, D), jnp.float32),
    )(x, degsrc.reshape(2 * 2 * NT, 1))

    agg = _scatter_kernel(z, src_all, dst_all)

    h = pl.pallas_call(
        _out_body,
        out_shape=jax.ShapeDtypeStruct((N, D), jnp.float32),
    )(agg, degdst.reshape(2 * NT, 1), W0, W1, b0.reshape(1, D),
      b1.reshape(1, D), prelu_a.reshape(1, 1))
    return h


# speedup vs baseline: 14.1646x; 1.0688x over previous
import functools

import jax
import jax.numpy as jnp
from jax import lax
from jax.experimental import pallas as pl
from jax.experimental.pallas import tpu as pltpu
from jax.experimental.pallas import tpu_sc as plsc

N = 10000
D = 128
E = 320000
NC = 2
NS = 16
EPC = 128
NCH = E // EPC
FT = 160
FL = 100
G = 8
HS = 10240
NP = 10240
TS = NP // NS

_mesh = plsc.VectorSubcoreMesh(core_axis_name="c", subcore_axis_name="s")


def _fill_f32(ref, rows, cols, value):
    v = jnp.full((16,), value, dtype=jnp.float32)
    nchunks = cols // 16

    def body(i, _):
        for k in range(nchunks):
            ref[i, pl.ds(16 * k, 16)] = v
        return 0

    lax.fori_loop(0, rows, body, 0)


@functools.partial(
    pl.kernel,
    out_type=(
        jax.ShapeDtypeStruct((NC, HS), jnp.float32),
        jax.ShapeDtypeStruct((NC, HS), jnp.float32),
    ),
    mesh=_mesh,
    scratch_types=[
        pltpu.VMEM((FT, EPC), jnp.int32),
        pltpu.VMEM((FT, EPC), jnp.int32),
        pltpu.VMEM((1, EPC), jnp.float32),
        pltpu.VMEM((1, HS // NS), jnp.float32),
        pltpu.VMEM_SHARED((HS,), jnp.float32),
        pltpu.VMEM_SHARED((HS,), jnp.float32),
        pltpu.SemaphoreType.DMA,
        pltpu.SemaphoreType.DMA,
    ],
)
def _deg_kernel(s0_hbm, d0_hbm, s1_hbm, d1_hbm, tails_hbm,
                degsrc_hbm, degdst_hbm,
                src_v, dst_v, ones_v, zeros_v, shist, dhist,
                sem0, sem1):
    c = lax.axis_index("c")
    s = lax.axis_index("s")
    hslice = HS // NS

    _fill_f32(ones_v, 1, EPC, 1.0)
    _fill_f32(zeros_v, 1, hslice, 0.0)
    pltpu.sync_copy(zeros_v.at[0], shist.at[pl.ds(s * hslice, hslice)])
    pltpu.sync_copy(zeros_v.at[0], dhist.at[pl.ds(s * hslice, hslice)])
    plsc.subcore_barrier()

    def run(src_hbm, dst_hbm, rel, f, tail):
        base = s * FT
        fa = f - 4 if tail else f
        pltpu.sync_copy(src_hbm.at[pl.ds(base, fa)], src_v.at[pl.ds(0, fa)])
        pltpu.sync_copy(dst_hbm.at[pl.ds(base, fa)], dst_v.at[pl.ds(0, fa)])
        if tail:
            for r in range(4):
                pltpu.sync_copy(
                    tails_hbm.at[pl.ds((2 * rel + 0) * 512 + EPC * r, EPC)],
                    src_v.at[fa + r])
                pltpu.sync_copy(
                    tails_hbm.at[pl.ds((2 * rel + 1) * 512 + EPC * r, EPC)],
                    dst_v.at[fa + r])

        def body(j, _):
            a = pltpu.async_copy(ones_v.at[0], shist.at[src_v.at[j]], sem0,
                                 add=True)
            b = pltpu.async_copy(ones_v.at[0], dhist.at[dst_v.at[j]], sem1,
                                 add=True)
            a.wait()
            b.wait()
            return 0

        lax.fori_loop(0, f, body, 0)

    @pl.when(jnp.logical_and(c == 0, s < NS - 1))
    def _():
        run(s0_hbm, d0_hbm, 0, FT, False)

    @pl.when(jnp.logical_and(c == 0, s == NS - 1))
    def _():
        run(s0_hbm, d0_hbm, 0, FL, True)

    @pl.when(jnp.logical_and(c == 1, s < NS - 1))
    def _():
        run(s1_hbm, d1_hbm, 1, FT, False)

    @pl.when(jnp.logical_and(c == 1, s == NS - 1))
    def _():
        run(s1_hbm, d1_hbm, 1, FL, True)

    plsc.subcore_barrier()
    pltpu.sync_copy(shist.at[pl.ds(s * hslice, hslice)],
                    degsrc_hbm.at[c, pl.ds(s * hslice, hslice)])
    pltpu.sync_copy(dhist.at[pl.ds(s * hslice, hslice)],
                    degdst_hbm.at[c, pl.ds(s * hslice, hslice)])


@functools.partial(
    pl.kernel,
    out_type=(
        jax.ShapeDtypeStruct((NP, D), jnp.float32),
        jax.ShapeDtypeStruct((NP, D), jnp.float32),
    ),
    mesh=_mesh,
    scratch_types=[
        pltpu.VMEM((2, G, EPC), jnp.int32),
        pltpu.VMEM((2, G, EPC), jnp.int32),
        pltpu.VMEM((EPC, D), jnp.float32),
        pltpu.VMEM((EPC, D), jnp.float32),
        pltpu.VMEM_SHARED((NP, D), jnp.float32),
        pltpu.SemaphoreType.DMA,
        pltpu.SemaphoreType.DMA,
        pltpu.SemaphoreType.DMA,
        pltpu.SemaphoreType.DMA,
        pltpu.SemaphoreType.DMA,
        pltpu.SemaphoreType.DMA,
    ],
)
def _scatter_kernel(z0_hbm, z1_hbm, s0_hbm, d0_hbm, s1_hbm, d1_hbm,
                    tails_hbm, agg0_hbm, agg1_hbm,
                    srcblk, dstblk, buf0, buf1, acc,
                    sem0, sem1, ssem0, ssem1, sem_si, sem_di):
    c = lax.axis_index("c")
    s = lax.axis_index("s")
    bufs = (buf0, buf1)
    sems = (sem0, sem1)
    ssems = (ssem0, ssem1)

    _fill_f32(buf0, EPC, D, 0.0)
    for k in range(TS // EPC):
        pltpu.sync_copy(buf0, acc.at[pl.ds(s * TS + k * EPC, EPC)])
    plsc.subcore_barrier()

    def run(z_hbm, src_hbm, dst_hbm, rel, nb, tail):
        base = s * FT
        f = nb * G

        def start_blk_load(q, p):
            pltpu.async_copy(src_hbm.at[pl.ds(base + G * q, G)],
                             srcblk.at[p], sem_si)
            pltpu.async_copy(dst_hbm.at[pl.ds(base + G * q, G)],
                             dstblk.at[p], sem_di)

        def wait_blk_load(q, p):
            pltpu.make_async_copy(src_hbm.at[pl.ds(base + G * q, G)],
                                  srcblk.at[p], sem_si).wait()
            pltpu.make_async_copy(dst_hbm.at[pl.ds(base + G * q, G)],
                                  dstblk.at[p], sem_di).wait()

        start_blk_load(0, 0)
        wait_blk_load(0, 0)
        start_blk_load(1, 1)
        pltpu.async_copy(z_hbm.at[srcblk.at[0, 0]], buf0, sem0)

        def gbody(u, _):
            for e in range(2):
                q = 2 * u + e
                for r in range(G):
                    b = r % 2
                    jj = G * q + r
                    if r == G - 1:
                        @pl.when(q + 1 < nb)
                        def _():
                            wait_blk_load(q + 1, 1 - e)

                    nxt_p = e if r < G - 1 else 1 - e
                    nxt_r = (r + 1) % G

                    @pl.when(jj >= 1)
                    def _():
                        pltpu.make_async_copy(bufs[1 - b],
                                              acc.at[dstblk.at[e, r]],
                                              ssems[1 - b]).wait()

                    @pl.when(jj + 1 < f)
                    def _():
                        pltpu.async_copy(z_hbm.at[srcblk.at[nxt_p, nxt_r]],
                                         bufs[1 - b], sems[1 - b])

                    pltpu.make_async_copy(z_hbm.at[srcblk.at[e, r]],
                                          bufs[b], sems[b]).wait()
                    pltpu.async_copy(bufs[b], acc.at[dstblk.at[e, r]],
                                     ssems[b], add=True)

                @pl.when(q + 2 < nb)
                def _():
                    start_blk_load(q + 2, e)
            return 0

        lax.fori_loop(0, nb // 2, gbody, 0)
        pltpu.make_async_copy(bufs[(f - 1) % 2], acc.at[dstblk.at[1, G - 1]],
                              ssems[(f - 1) % 2]).wait()

        if tail:
            for r in range(4):
                pltpu.sync_copy(
                    tails_hbm.at[pl.ds((2 * rel + 0) * 512 + EPC * r, EPC)],
                    srcblk.at[0, r])
                pltpu.sync_copy(
                    tails_hbm.at[pl.ds((2 * rel + 1) * 512 + EPC * r, EPC)],
                    dstblk.at[0, r])
            for r in range(4):
                pltpu.async_copy(z_hbm.at[srcblk.at[0, r]], buf0,
                                 sem0).wait()
                pltpu.sync_copy(buf0, acc.at[dstblk.at[0, r]], add=True)

    @pl.when(jnp.logical_and(c == 0, s < NS - 1))
    def _():
        run(z0_hbm, s0_hbm, d0_hbm, 0, FT // G, False)

    @pl.when(jnp.logical_and(c == 0, s == NS - 1))
    def _():
        run(z0_hbm, s0_hbm, d0_hbm, 0, (FL - 4) // G, True)

    @pl.when(jnp.logical_and(c == 1, s < NS - 1))
    def _():
        run(z1_hbm, s1_hbm, d1_hbm, 1, FT // G, False)

    @pl.when(jnp.logical_and(c == 1, s == NS - 1))
    def _():
        run(z1_hbm, s1_hbm, d1_hbm, 1, (FL - 4) // G, True)

    plsc.subcore_barrier()

    @pl.when(c == 0)
    def _():
        pltpu.sync_copy(acc.at[pl.ds(s * TS, TS)],
                        agg0_hbm.at[pl.ds(s * TS, TS)])

    @pl.when(c == 1)
    def _():
        pltpu.sync_copy(acc.at[pl.ds(s * TS, TS)],
                        agg1_hbm.at[pl.ds(s * TS, TS)])


def _mask_rsqrt(d):
    return jnp.where(d > 0, lax.rsqrt(jnp.maximum(d, 1.0)), 0.0)


def _scale_body(x_ref, degsrc_ref, z0_ref, z1_ref):
    x = x_ref[...]
    ns0 = _mask_rsqrt(degsrc_ref[pl.ds(0, N), :])
    ns1 = _mask_rsqrt(degsrc_ref[pl.ds(HS, N), :])
    z0_ref[...] = x * ns0
    z1_ref[...] = x * ns1


def _out_body(agg0_ref, agg1_ref, degdst_ref, w0_ref, w1_ref, b0_ref,
              b1_ref, a_ref, h_ref):
    nd0 = _mask_rsqrt(degdst_ref[pl.ds(0, N), :])
    nd1 = _mask_rsqrt(degdst_ref[pl.ds(HS, N), :])
    m0 = jnp.dot(agg0_ref[pl.ds(0, N), :], w0_ref[...],
                 preferred_element_type=jnp.float32)
    m1 = jnp.dot(agg1_ref[pl.ds(0, N), :], w1_ref[...],
                 preferred_element_type=jnp.float32)
    h = nd0 * m0 + nd1 * m1 + (b0_ref[...] + b1_ref[...])
    a = a_ref[0, 0]
    h_ref[...] = jnp.where(h > 0, h, a * h)


def kernel(x, edge_index_rel0, edge_index_rel1, W0, b0, W1, b1, prelu_a):
    s0 = edge_index_rel0[0].reshape(NCH, EPC)
    d0 = edge_index_rel0[1].reshape(NCH, EPC)
    s1 = edge_index_rel1[0].reshape(NCH, EPC)
    d1 = edge_index_rel1[1].reshape(NCH, EPC)

    t0 = (NS - 1) * FT + (FL - 4)
    tails = jnp.concatenate([
        edge_index_rel0[0][t0 * EPC:],
        edge_index_rel0[1][t0 * EPC:],
        edge_index_rel1[0][t0 * EPC:],
        edge_index_rel1[1][t0 * EPC:],
    ])

    degsrc, degdst = _deg_kernel(s0, d0, s1, d1, tails)

    z0, z1 = pl.pallas_call(
        _scale_body,
        out_shape=(jax.ShapeDtypeStruct((N, D), jnp.float32),
                   jax.ShapeDtypeStruct((N, D), jnp.float32)),
    )(x, degsrc.reshape(NC * HS, 1))

    agg0, agg1 = _scatter_kernel(z0, z1, s0, d0, s1, d1, tails)

    h = pl.pallas_call(
        _out_body,
        out_shape=jax.ShapeDtypeStruct((N, D), jnp.float32),
    )(agg0, agg1, degdst.reshape(NC * HS, 1), W0, W1, b0.reshape(1, D),
      b1.reshape(1, D), prelu_a.reshape(1, 1))
    return h


# speedup vs baseline: 14.3832x; 1.0154x over previous
import functools

import jax
import jax.numpy as jnp
from jax import lax
from jax.experimental import pallas as pl
from jax.experimental.pallas import tpu as pltpu
from jax.experimental.pallas import tpu_sc as plsc

N = 10000
D = 128
E = 320000
NC = 2
NS = 16
EPC = 128
NCH = E // EPC
FT = 160
FL = 100
G = 8
HS = 10240
NP = 10240
TS = NP // NS

_mesh = plsc.VectorSubcoreMesh(core_axis_name="c", subcore_axis_name="s")


def _fill_f32(ref, rows, cols, value):
    v = jnp.full((16,), value, dtype=jnp.float32)
    nchunks = cols // 16

    def body(i, _):
        for k in range(nchunks):
            ref[i, pl.ds(16 * k, 16)] = v
        return 0

    lax.fori_loop(0, rows, body, 0)


@functools.partial(
    pl.kernel,
    out_type=(
        jax.ShapeDtypeStruct((NC, HS), jnp.float32),
        jax.ShapeDtypeStruct((NC, HS), jnp.float32),
    ),
    mesh=_mesh,
    scratch_types=[
        pltpu.VMEM((FT, EPC), jnp.int32),
        pltpu.VMEM((FT, EPC), jnp.int32),
        pltpu.VMEM((1, EPC), jnp.float32),
        pltpu.VMEM((1, HS // NS), jnp.float32),
        pltpu.VMEM_SHARED((HS,), jnp.float32),
        pltpu.VMEM_SHARED((HS,), jnp.float32),
        pltpu.SemaphoreType.DMA,
        pltpu.SemaphoreType.DMA,
        pltpu.SemaphoreType.DMA,
        pltpu.SemaphoreType.DMA,
    ],
)
def _deg_kernel(s0_hbm, d0_hbm, s1_hbm, d1_hbm, tails_hbm,
                degsrc_hbm, degdst_hbm,
                src_v, dst_v, ones_v, zeros_v, shist, dhist,
                sem0, sem1, sem2, sem3):
    c = lax.axis_index("c")
    s = lax.axis_index("s")
    hslice = HS // NS

    _fill_f32(ones_v, 1, EPC, 1.0)
    _fill_f32(zeros_v, 1, hslice, 0.0)
    pltpu.sync_copy(zeros_v.at[0], shist.at[pl.ds(s * hslice, hslice)])
    pltpu.sync_copy(zeros_v.at[0], dhist.at[pl.ds(s * hslice, hslice)])
    plsc.subcore_barrier()

    def run(src_hbm, dst_hbm, rel, f, tail):
        base = s * FT
        fa = f - 4 if tail else f
        pltpu.sync_copy(src_hbm.at[pl.ds(base, fa)], src_v.at[pl.ds(0, fa)])
        pltpu.sync_copy(dst_hbm.at[pl.ds(base, fa)], dst_v.at[pl.ds(0, fa)])
        if tail:
            for r in range(4):
                pltpu.sync_copy(
                    tails_hbm.at[pl.ds((2 * rel + 0) * 512 + EPC * r, EPC)],
                    src_v.at[fa + r])
                pltpu.sync_copy(
                    tails_hbm.at[pl.ds((2 * rel + 1) * 512 + EPC * r, EPC)],
                    dst_v.at[fa + r])

        ssem = (sem0, sem1)
        dsem = (sem2, sem3)

        def start_pair(j, b):
            pltpu.async_copy(ones_v.at[0], shist.at[src_v.at[j]], ssem[b],
                             add=True)
            pltpu.async_copy(ones_v.at[0], dhist.at[dst_v.at[j]], dsem[b],
                             add=True)

        def wait_pair(j, b):
            pltpu.make_async_copy(ones_v.at[0], shist.at[src_v.at[j]],
                                  ssem[b]).wait()
            pltpu.make_async_copy(ones_v.at[0], dhist.at[dst_v.at[j]],
                                  dsem[b]).wait()

        start_pair(0, 0)

        def body(u, _):
            for b in range(2):
                j = 2 * u + b

                @pl.when(j + 1 < f)
                def _():
                    start_pair(j + 1, 1 - b)

                wait_pair(j, b)
            return 0

        lax.fori_loop(0, f // 2, body, 0)

    @pl.when(jnp.logical_and(c == 0, s < NS - 1))
    def _():
        run(s0_hbm, d0_hbm, 0, FT, False)

    @pl.when(jnp.logical_and(c == 0, s == NS - 1))
    def _():
        run(s0_hbm, d0_hbm, 0, FL, True)

    @pl.when(jnp.logical_and(c == 1, s < NS - 1))
    def _():
        run(s1_hbm, d1_hbm, 1, FT, False)

    @pl.when(jnp.logical_and(c == 1, s == NS - 1))
    def _():
        run(s1_hbm, d1_hbm, 1, FL, True)

    plsc.subcore_barrier()
    pltpu.sync_copy(shist.at[pl.ds(s * hslice, hslice)],
                    degsrc_hbm.at[c, pl.ds(s * hslice, hslice)])
    pltpu.sync_copy(dhist.at[pl.ds(s * hslice, hslice)],
                    degdst_hbm.at[c, pl.ds(s * hslice, hslice)])


@functools.partial(
    pl.kernel,
    out_type=(
        jax.ShapeDtypeStruct((NP, D), jnp.float32),
        jax.ShapeDtypeStruct((NP, D), jnp.float32),
    ),
    mesh=_mesh,
    scratch_types=[
        pltpu.VMEM((2, G, EPC), jnp.int32),
        pltpu.VMEM((2, G, EPC), jnp.int32),
        pltpu.VMEM((EPC, D), jnp.float32),
        pltpu.VMEM((EPC, D), jnp.float32),
        pltpu.VMEM_SHARED((NP, D), jnp.float32),
        pltpu.SemaphoreType.DMA,
        pltpu.SemaphoreType.DMA,
        pltpu.SemaphoreType.DMA,
        pltpu.SemaphoreType.DMA,
        pltpu.SemaphoreType.DMA,
        pltpu.SemaphoreType.DMA,
    ],
)
def _scatter_kernel(z0_hbm, z1_hbm, s0_hbm, d0_hbm, s1_hbm, d1_hbm,
                    tails_hbm, agg0_hbm, agg1_hbm,
                    srcblk, dstblk, buf0, buf1, acc,
                    sem0, sem1, ssem0, ssem1, sem_si, sem_di):
    c = lax.axis_index("c")
    s = lax.axis_index("s")
    bufs = (buf0, buf1)
    sems = (sem0, sem1)
    ssems = (ssem0, ssem1)

    _fill_f32(buf0, EPC, D, 0.0)
    for k in range(TS // EPC):
        pltpu.sync_copy(buf0, acc.at[pl.ds(s * TS + k * EPC, EPC)])
    plsc.subcore_barrier()

    def run(z_hbm, src_hbm, dst_hbm, rel, nb, tail):
        base = s * FT
        f = nb * G

        def start_blk_load(q, p):
            pltpu.async_copy(src_hbm.at[pl.ds(base + G * q, G)],
                             srcblk.at[p], sem_si)
            pltpu.async_copy(dst_hbm.at[pl.ds(base + G * q, G)],
                             dstblk.at[p], sem_di)

        def wait_blk_load(q, p):
            pltpu.make_async_copy(src_hbm.at[pl.ds(base + G * q, G)],
                                  srcblk.at[p], sem_si).wait()
            pltpu.make_async_copy(dst_hbm.at[pl.ds(base + G * q, G)],
                                  dstblk.at[p], sem_di).wait()

        start_blk_load(0, 0)
        wait_blk_load(0, 0)
        start_blk_load(1, 1)
        pltpu.async_copy(z_hbm.at[srcblk.at[0, 0]], buf0, sem0)

        def gbody(u, _):
            for e in range(2):
                q = 2 * u + e
                for r in range(G):
                    b = r % 2
                    jj = G * q + r
                    if r == G - 1:
                        @pl.when(q + 1 < nb)
                        def _():
                            wait_blk_load(q + 1, 1 - e)

                    nxt_p = e if r < G - 1 else 1 - e
                    nxt_r = (r + 1) % G

                    @pl.when(jj >= 1)
                    def _():
                        pltpu.make_async_copy(bufs[1 - b],
                                              acc.at[dstblk.at[e, r]],
                                              ssems[1 - b]).wait()

                    @pl.when(jj + 1 < f)
                    def _():
                        pltpu.async_copy(z_hbm.at[srcblk.at[nxt_p, nxt_r]],
                                         bufs[1 - b], sems[1 - b])

                    pltpu.make_async_copy(z_hbm.at[srcblk.at[e, r]],
                                          bufs[b], sems[b]).wait()
                    pltpu.async_copy(bufs[b], acc.at[dstblk.at[e, r]],
                                     ssems[b], add=True)

                @pl.when(q + 2 < nb)
                def _():
                    start_blk_load(q + 2, e)
            return 0

        lax.fori_loop(0, nb // 2, gbody, 0)
        pltpu.make_async_copy(bufs[(f - 1) % 2], acc.at[dstblk.at[1, G - 1]],
                              ssems[(f - 1) % 2]).wait()

        if tail:
            for r in range(4):
                pltpu.sync_copy(
                    tails_hbm.at[pl.ds((2 * rel + 0) * 512 + EPC * r, EPC)],
                    srcblk.at[0, r])
                pltpu.sync_copy(
                    tails_hbm.at[pl.ds((2 * rel + 1) * 512 + EPC * r, EPC)],
                    dstblk.at[0, r])
            for r in range(4):
                pltpu.async_copy(z_hbm.at[srcblk.at[0, r]], buf0,
                                 sem0).wait()
                pltpu.sync_copy(buf0, acc.at[dstblk.at[0, r]], add=True)

    @pl.when(jnp.logical_and(c == 0, s < NS - 1))
    def _():
        run(z0_hbm, s0_hbm, d0_hbm, 0, FT // G, False)

    @pl.when(jnp.logical_and(c == 0, s == NS - 1))
    def _():
        run(z0_hbm, s0_hbm, d0_hbm, 0, (FL - 4) // G, True)

    @pl.when(jnp.logical_and(c == 1, s < NS - 1))
    def _():
        run(z1_hbm, s1_hbm, d1_hbm, 1, FT // G, False)

    @pl.when(jnp.logical_and(c == 1, s == NS - 1))
    def _():
        run(z1_hbm, s1_hbm, d1_hbm, 1, (FL - 4) // G, True)

    plsc.subcore_barrier()

    @pl.when(c == 0)
    def _():
        pltpu.sync_copy(acc.at[pl.ds(s * TS, TS)],
                        agg0_hbm.at[pl.ds(s * TS, TS)])

    @pl.when(c == 1)
    def _():
        pltpu.sync_copy(acc.at[pl.ds(s * TS, TS)],
                        agg1_hbm.at[pl.ds(s * TS, TS)])


def _mask_rsqrt(d):
    return jnp.where(d > 0, lax.rsqrt(jnp.maximum(d, 1.0)), 0.0)


BR = 1000


def _scale_body(x_ref, degsrc_ref, z0_ref, z1_ref):
    i = pl.program_id(0)
    r0 = pl.multiple_of(i * BR, 8)
    r1 = pl.multiple_of(HS + i * BR, 8)
    x = x_ref[...]
    ns0 = _mask_rsqrt(degsrc_ref[pl.ds(r0, BR), :])
    ns1 = _mask_rsqrt(degsrc_ref[pl.ds(r1, BR), :])
    z0_ref[...] = x * ns0
    z1_ref[...] = x * ns1


def _out_body(agg0_ref, agg1_ref, degdst_ref, w0_ref, w1_ref, b0_ref,
              b1_ref, a_ref, h_ref):
    i = pl.program_id(0)
    r0 = pl.multiple_of(i * BR, 8)
    r1 = pl.multiple_of(HS + i * BR, 8)
    nd0 = _mask_rsqrt(degdst_ref[pl.ds(r0, BR), :])
    nd1 = _mask_rsqrt(degdst_ref[pl.ds(r1, BR), :])
    m0 = jnp.dot(agg0_ref[...], w0_ref[...],
                 preferred_element_type=jnp.float32)
    m1 = jnp.dot(agg1_ref[...], w1_ref[...],
                 preferred_element_type=jnp.float32)
    h = nd0 * m0 + nd1 * m1 + (b0_ref[...] + b1_ref[...])
    a = a_ref[0, 0]
    h_ref[...] = jnp.where(h > 0, h, a * h)


def kernel(x, edge_index_rel0, edge_index_rel1, W0, b0, W1, b1, prelu_a):
    s0 = edge_index_rel0[0].reshape(NCH, EPC)
    d0 = edge_index_rel0[1].reshape(NCH, EPC)
    s1 = edge_index_rel1[0].reshape(NCH, EPC)
    d1 = edge_index_rel1[1].reshape(NCH, EPC)

    t0 = (NS - 1) * FT + (FL - 4)
    tails = jnp.concatenate([
        edge_index_rel0[0][t0 * EPC:],
        edge_index_rel0[1][t0 * EPC:],
        edge_index_rel1[0][t0 * EPC:],
        edge_index_rel1[1][t0 * EPC:],
    ])

    degsrc, degdst = _deg_kernel(s0, d0, s1, d1, tails)

    z0, z1 = pl.pallas_call(
        _scale_body,
        grid=(N // BR,),
        in_specs=[pl.BlockSpec((BR, D), lambda i: (i, 0)),
                  pl.BlockSpec((NC * HS, 1), lambda i: (0, 0))],
        out_specs=(pl.BlockSpec((BR, D), lambda i: (i, 0)),
                   pl.BlockSpec((BR, D), lambda i: (i, 0))),
        out_shape=(jax.ShapeDtypeStruct((N, D), jnp.float32),
                   jax.ShapeDtypeStruct((N, D), jnp.float32)),
    )(x, degsrc.reshape(NC * HS, 1))

    agg0, agg1 = _scatter_kernel(z0, z1, s0, d0, s1, d1, tails)

    h = pl.pallas_call(
        _out_body,
        grid=(N // BR,),
        in_specs=[pl.BlockSpec((BR, D), lambda i: (i, 0)),
                  pl.BlockSpec((BR, D), lambda i: (i, 0)),
                  pl.BlockSpec((NC * HS, 1), lambda i: (0, 0)),
                  pl.BlockSpec((D, D), lambda i: (0, 0)),
                  pl.BlockSpec((D, D), lambda i: (0, 0)),
                  pl.BlockSpec((1, D), lambda i: (0, 0)),
                  pl.BlockSpec((1, D), lambda i: (0, 0)),
                  pl.BlockSpec((1, 1), lambda i: (0, 0))],
        out_specs=pl.BlockSpec((BR, D), lambda i: (i, 0)),
        out_shape=jax.ShapeDtypeStruct((N, D), jnp.float32),
    )(agg0, agg1, degdst.reshape(NC * HS, 1), W0, W1, b0.reshape(1, D),
      b1.reshape(1, D), prelu_a.reshape(1, 1))
    return h
